# baseline (speedup 1.0000x reference)
import functools

import jax
import jax.numpy as jnp
from jax import lax
from jax.experimental import pallas as pl
from jax.experimental.pallas import tpu as pltpu
from jax.experimental.pallas import tpu_sc as plsc

N_NODES = 10000
NPAD = 10240
D1 = 128
D2 = 64
NC, NS, L = 2, 16, 16
NW = NC * NS
RPT = NPAD // NS
E_EDGES = 320000
CH = 128
EPT = ((E_EDGES // NW + CH * 8 - 1) // (CH * 8)) * (CH * 8)
NCHUNK = EPT // CH
EROWS = NW * NCHUNK
SB = 8

_MESH = dict(core_axis_name="c", subcore_axis_name="s", num_cores=NC,
             num_subcores=NS)
_SC_PARAMS = pltpu.CompilerParams(needs_layout_passes=False,
                                  use_tc_tiling_on_sc=False)


def _deg_body(dst_hbm, ew_hbm, degp_hbm, dst_v, ew_v, hist_v, red_v, slab_sh):
    c = lax.axis_index("c")
    s = lax.axis_index("s")
    wid = c * NS + s
    base = wid * NCHUNK
    pltpu.sync_copy(dst_hbm.at[pl.ds(base, NCHUNK)], dst_v)
    pltpu.sync_copy(ew_hbm.at[pl.ds(base, NCHUNK)], ew_v)

    def zero(i, _):
        hist_v[pl.ds(i * L, L)] = jnp.zeros((L,), jnp.float32)
        return 0
    lax.fori_loop(0, NPAD // L, zero, 0)

    def acc(i, _):
        for g in range(CH // L):
            dv = dst_v[i, pl.ds(g * L, L)]
            wv = ew_v[i, pl.ds(g * L, L)]
            plsc.addupdate_scatter(hist_v, [dv], wv)
        return 0
    lax.fori_loop(0, NCHUNK, acc, 0)

    pltpu.sync_copy(hist_v, slab_sh.at[s])
    plsc.subcore_barrier()
    pltpu.sync_copy(slab_sh.at[pl.ds(0, NS), pl.ds(s * RPT, RPT)], red_v)

    def red(i, _):
        v = red_v[0, pl.ds(i * L, L)]
        for p in range(1, NS):
            v = v + red_v[p, pl.ds(i * L, L)]
        hist_v[pl.ds(i * L, L)] = v
        return 0
    lax.fori_loop(0, RPT // L, red, 0)
    pltpu.sync_copy(hist_v.at[pl.ds(0, RPT)],
                    degp_hbm.at[c, pl.ds(s * RPT, RPT)])


def _degree_partials(dst2, ew2):
    return pl.kernel(
        _deg_body,
        out_type=jax.ShapeDtypeStruct((NC, NPAD), jnp.float32),
        mesh=plsc.VectorSubcoreMesh(**_MESH),
        compiler_params=_SC_PARAMS,
        scratch_types=[
            pltpu.VMEM((NCHUNK, CH), jnp.int32),
            pltpu.VMEM((NCHUNK, CH), jnp.float32),
            pltpu.VMEM((NPAD,), jnp.float32),
            pltpu.VMEM((NS, RPT), jnp.float32),
            pltpu.VMEM_SHARED((NS, NPAD), jnp.float32),
        ],
    )(dst2, ew2)


NSUP = NCHUNK // SB


def _msg128_body(D, src_hbm, dst_hbm, ew_hbm, dis_hbm, xw_hbm, part_hbm,
              src_v, dst_v, ew_v, dis_v, nrm_v, rows0_v, rows1_v,
              g0, g1, s0, s1, acc_sh):
    c = lax.axis_index("c")
    s = lax.axis_index("s")
    wid = c * NS + s
    base = wid * NCHUNK
    rows = (rows0_v, rows1_v)
    gsem = (g0, g1)
    ssem = (s0, s1)
    pltpu.sync_copy(dis_hbm, dis_v)

    def zrow(i, _):
        for g in range(D // L):
            rows0_v[i, pl.ds(g * L, L)] = jnp.zeros((L,), jnp.float32)
        return 0
    lax.fori_loop(0, CH, zrow, 0)
    for k in range(RPT // CH):
        pltpu.sync_copy(rows0_v, acc_sh.at[pl.ds(s * RPT + k * CH, CH)])
    plsc.subcore_barrier()

    def stage(sjn):
        pltpu.sync_copy(src_hbm.at[pl.ds(base + sjn * SB, SB)], src_v)
        pltpu.sync_copy(ew_hbm.at[pl.ds(base + sjn * SB, SB)], ew_v)
        pltpu.sync_copy(dst_hbm.at[pl.ds(base + sjn * SB, SB)],
                        dst_v.at[sjn % 2])

    stage(0)
    pltpu.async_copy(xw_hbm.at[src_v.at[0]], rows0_v, g0)

    def superchunk(sj, _):
        m = sj % 2
        for k in range(SB):
            buf = k % 2
            nbuf = 1 - buf
            pltpu.make_async_copy(xw_hbm.at[src_v.at[k]], rows[buf],
                                  gsem[buf]).wait()
            for g in range(CH // L):
                sv = src_v[k, pl.ds(g * L, L)]
                dv = dst_v[m, k, pl.ds(g * L, L)]
                wv = ew_v[k, pl.ds(g * L, L)]
                nrm_v[pl.ds(g * L, L)] = (plsc.load_gather(dis_v, [sv]) * wv
                                          * plsc.load_gather(dis_v, [dv]))

            def _wait_prev_scatter():
                pltpu.make_async_copy(rows[nbuf], acc_sh.at[dst_v.at[m, k]],
                                      ssem[nbuf]).wait()
            if k == SB - 1:
                @pl.when(sj + 1 < NSUP)
                def _():
                    stage(sj + 1)
                    _wait_prev_scatter()
                    pltpu.async_copy(xw_hbm.at[src_v.at[0]], rows[nbuf],
                                     gsem[nbuf])
            elif k == 0:
                @pl.when(sj > 0)
                def _():
                    _wait_prev_scatter()
                pltpu.async_copy(xw_hbm.at[src_v.at[k + 1]], rows[nbuf],
                                 gsem[nbuf])
            else:
                _wait_prev_scatter()
                pltpu.async_copy(xw_hbm.at[src_v.at[k + 1]], rows[nbuf],
                                 gsem[nbuf])

            def scale(q, _):
                nv = nrm_v[pl.ds(q * L, L)]
                for t in range(L):
                    sc = nv[t]
                    e = q * L + t
                    for g in range(D // L):
                        rows[buf][e, pl.ds(g * L, L)] = (
                            rows[buf][e, pl.ds(g * L, L)] * sc)
                return 0
            lax.fori_loop(0, CH // L, scale, 0)
            pltpu.async_copy(rows[buf], acc_sh.at[dst_v.at[m, k]],
                             ssem[buf], add=True)
        return 0
    lax.fori_loop(0, NSUP, superchunk, 0)
    mlast = (NSUP - 1) % 2
    for b in range(2):
        pltpu.make_async_copy(rows[b], acc_sh.at[dst_v.at[mlast, SB - 2 + b]],
                              ssem[b]).wait()
    plsc.subcore_barrier()
    pltpu.sync_copy(acc_sh.at[pl.ds(s * RPT, RPT)],
                    part_hbm.at[c, pl.ds(s * RPT, RPT)])


def _message_partials128(src2, dst2, ew2, dis, xw):
    return pl.kernel(
        functools.partial(_msg128_body, D1),
        out_type=jax.ShapeDtypeStruct((NC, NPAD, D1), jnp.float32),
        mesh=plsc.VectorSubcoreMesh(**_MESH),
        compiler_params=_SC_PARAMS,
        scratch_types=[
            pltpu.VMEM((SB, CH), jnp.int32),
            pltpu.VMEM((2, SB, CH), jnp.int32),
            pltpu.VMEM((SB, CH), jnp.float32),
            pltpu.VMEM((NPAD,), jnp.float32),
            pltpu.VMEM((CH,), jnp.float32),
            pltpu.VMEM((CH, D1), jnp.float32),
            pltpu.VMEM((CH, D1), jnp.float32),
            pltpu.SemaphoreType.DMA,
            pltpu.SemaphoreType.DMA,
            pltpu.SemaphoreType.DMA,
            pltpu.SemaphoreType.DMA,
            pltpu.VMEM_SHARED((NPAD, D1), jnp.float32),
        ],
    )(src2, dst2, ew2, dis, xw)


CH2 = 256
NBC = EPT // CH2
EROWS2 = NW * NBC
CPS = 4
SB2 = CPS


def _msg_body(src_hbm, dst_hbm, ew_hbm, dis_hbm, xw_hbm,
              part_hbm, src_v, dst_v, ew_v, dis_v, nrm_v, rows0_v, rows1_v,
              g0, g1, s0, s1, acc_sh, xw_sh):
    D = D2
    c = lax.axis_index("c")
    s = lax.axis_index("s")
    wid = c * NS + s
    base = wid * NBC
    rows = (rows0_v, rows1_v)
    gsem = (g0, g1)
    ssem = (s0, s1)
    pltpu.sync_copy(dis_hbm, dis_v)
    pltpu.sync_copy(xw_hbm.at[pl.ds(s * RPT, RPT)],
                    xw_sh.at[pl.ds(s * RPT, RPT)])

    def zrow(i, _):
        for g in range(D // L):
            rows0_v[i, pl.ds(g * L, L)] = jnp.zeros((L,), jnp.float32)
        return 0
    lax.fori_loop(0, CH2, zrow, 0)
    for k in range(RPT // CH2):
        pltpu.sync_copy(rows0_v, acc_sh.at[pl.ds(s * RPT + k * CH2, CH2)])
    pltpu.sync_copy(rows0_v.at[pl.ds(0, RPT % CH2)],
                    acc_sh.at[pl.ds(s * RPT + (RPT // CH2) * CH2,
                                    RPT % CH2)])
    plsc.subcore_barrier()

    def stage(sjn):
        pltpu.sync_copy(src_hbm.at[pl.ds(base + sjn * SB2, SB2)], src_v)
        pltpu.sync_copy(ew_hbm.at[pl.ds(base + sjn * SB2, SB2)], ew_v)
        pltpu.sync_copy(dst_hbm.at[pl.ds(base + sjn * SB2, SB2)],
                        dst_v.at[sjn % 2])

    stage(0)
    pltpu.async_copy(xw_sh.at[src_v.at[0]], rows0_v, g0)

    def superchunk(sj, _):
        m = sj % 2
        for t in range(CPS):
            buf = t % 2
            nbuf = 1 - buf
            pltpu.make_async_copy(xw_sh.at[src_v.at[t]],
                                  rows[buf], gsem[buf]).wait()
            for g in range(CH2 // L):
                col = g * L
                sv = src_v[t, pl.ds(col, L)]
                dv = dst_v[m, t, pl.ds(col, L)]
                wv = ew_v[t, pl.ds(col, L)]
                nrm_v[pl.ds(g * L, L)] = (plsc.load_gather(dis_v, [sv]) * wv
                                          * plsc.load_gather(dis_v, [dv]))

            def _wait_prev_scatter():
                pltpu.make_async_copy(
                    rows[nbuf], acc_sh.at[dst_v.at[m, t]],
                    ssem[nbuf]).wait()
            if t == CPS - 1:
                @pl.when(sj + 1 < NSUP)
                def _():
                    stage(sj + 1)
                    _wait_prev_scatter()
                    pltpu.async_copy(xw_sh.at[src_v.at[0]],
                                     rows[nbuf], gsem[nbuf])
            elif t == 0:
                @pl.when(sj > 0)
                def _():
                    _wait_prev_scatter()
                pltpu.async_copy(xw_sh.at[src_v.at[t + 1]],
                                 rows[nbuf], gsem[nbuf])
            else:
                _wait_prev_scatter()
                pltpu.async_copy(xw_sh.at[src_v.at[t + 1]],
                                 rows[nbuf], gsem[nbuf])

            def scale(q, _):
                nv = nrm_v[pl.ds(q * L, L)]
                for tt in range(L):
                    sc = nv[tt]
                    e = q * L + tt
                    for g in range(D // L):
                        rows[buf][e, pl.ds(g * L, L)] = (
                            rows[buf][e, pl.ds(g * L, L)] * sc)
                return 0
            lax.fori_loop(0, CH2 // L, scale, 0)
            pltpu.async_copy(rows[buf],
                             acc_sh.at[dst_v.at[m, t]],
                             ssem[buf], add=True)
        return 0
    lax.fori_loop(0, NSUP, superchunk, 0)
    mlast = (NSUP - 1) % 2
    for b in range(2):
        pltpu.make_async_copy(
            rows[b], acc_sh.at[dst_v.at[mlast, CPS - 2 + b]],
            ssem[b]).wait()
    plsc.subcore_barrier()
    pltpu.sync_copy(acc_sh.at[pl.ds(s * RPT, RPT)],
                    part_hbm.at[c, pl.ds(s * RPT, RPT)])


def _message_partials(src2, dst2, ew2, dis, xw):
    return pl.kernel(
        _msg_body,
        out_type=jax.ShapeDtypeStruct((NC, NPAD, D2), jnp.float32),
        mesh=plsc.VectorSubcoreMesh(**_MESH),
        compiler_params=_SC_PARAMS,
        scratch_types=[
            pltpu.VMEM((SB2, CH2), jnp.int32),
            pltpu.VMEM((2, SB2, CH2), jnp.int32),
            pltpu.VMEM((SB2, CH2), jnp.float32),
            pltpu.VMEM((NPAD,), jnp.float32),
            pltpu.VMEM((CH2,), jnp.float32),
            pltpu.VMEM((CH2, D2), jnp.float32),
            pltpu.VMEM((CH2, D2), jnp.float32),
            pltpu.SemaphoreType.DMA,
            pltpu.SemaphoreType.DMA,
            pltpu.SemaphoreType.DMA,
            pltpu.SemaphoreType.DMA,
            pltpu.VMEM_SHARED((NPAD, D2), jnp.float32),
            pltpu.VMEM_SHARED((NPAD, D2), jnp.float32),
        ],
    )(src2, dst2, ew2, dis, xw)


_BR = 1024


def _tc1_body(x_ref, w_ref, degp_ref, xw_ref, dis_ref, dis2_ref):
    z = jnp.dot(x_ref[...], w_ref[...], preferred_element_type=jnp.float32)
    xw_ref[...] = z
    deg = degp_ref[0] + degp_ref[1] + 1.0
    d = jnp.where(deg > 0, lax.rsqrt(deg), 0.0)
    dis_ref[...] = d
    dis2_ref[...] = d * d


def _tc1(x_pad, W1, degp3):
    grid = (NPAD // _BR,)
    return pl.pallas_call(
        _tc1_body,
        grid=grid,
        in_specs=[
            pl.BlockSpec((_BR, D1), lambda i: (i, 0)),
            pl.BlockSpec((D1, D1), lambda i: (0, 0)),
            pl.BlockSpec((NC, _BR, 1), lambda i: (0, i, 0)),
        ],
        out_specs=[
            pl.BlockSpec((_BR, D1), lambda i: (i, 0)),
            pl.BlockSpec((_BR, 1), lambda i: (i, 0)),
            pl.BlockSpec((_BR, 1), lambda i: (i, 0)),
        ],
        out_shape=[
            jax.ShapeDtypeStruct((NPAD, D1), jnp.float32),
            jax.ShapeDtypeStruct((NPAD, 1), jnp.float32),
            jax.ShapeDtypeStruct((NPAD, 1), jnp.float32),
        ],
    )(x_pad, W1, degp3)


def _tc2_body(p_ref, xw_ref, dis2_ref, b_ref, w2_ref, xw2_ref):
    x1 = p_ref[0] + p_ref[1] + xw_ref[...] * dis2_ref[...] + b_ref[...]
    x1 = jnp.maximum(x1, 0.0)
    xw2_ref[...] = jnp.dot(x1, w2_ref[...], preferred_element_type=jnp.float32)


def _tc2(part1, xw1, dis2, b1r, W2):
    grid = (NPAD // _BR,)
    return pl.pallas_call(
        _tc2_body,
        grid=grid,
        in_specs=[
            pl.BlockSpec((NC, _BR, D1), lambda i: (0, i, 0)),
            pl.BlockSpec((_BR, D1), lambda i: (i, 0)),
            pl.BlockSpec((_BR, 1), lambda i: (i, 0)),
            pl.BlockSpec((1, D1), lambda i: (0, 0)),
            pl.BlockSpec((D1, D2), lambda i: (0, 0)),
        ],
        out_specs=pl.BlockSpec((_BR, D2), lambda i: (i, 0)),
        out_shape=jax.ShapeDtypeStruct((NPAD, D2), jnp.float32),
    )(part1, xw1, dis2, b1r, W2)


def _tc3_body(p_ref, xw_ref, dis2_ref, b_ref, wh_ref, bh_ref, mult_ref,
              lo_ref, hi_ref, out_ref):
    x2 = p_ref[0] + p_ref[1] + xw_ref[...] * dis2_ref[...] + b_ref[...]
    x2 = jnp.maximum(x2, 0.0)
    z = jnp.dot(x2, wh_ref[...], preferred_element_type=jnp.float32)
    z = z + bh_ref[...]
    h = 1.0 / (1.0 + jnp.exp(-z))
    out_ref[...] = jnp.clip(h * mult_ref[...], lo_ref[...], hi_ref[...])


def _tc3(part2, xw2, dis2, b2r, Wh, bhr, mult, lo, hi):
    grid = (NPAD // _BR,)
    return pl.pallas_call(
        _tc3_body,
        grid=grid,
        in_specs=[
            pl.BlockSpec((NC, _BR, D2), lambda i: (0, i, 0)),
            pl.BlockSpec((_BR, D2), lambda i: (i, 0)),
            pl.BlockSpec((_BR, 1), lambda i: (i, 0)),
            pl.BlockSpec((1, D2), lambda i: (0, 0)),
            pl.BlockSpec((D2, D1), lambda i: (0, 0)),
            pl.BlockSpec((1, D1), lambda i: (0, 0)),
            pl.BlockSpec((1, D1), lambda i: (0, 0)),
            pl.BlockSpec((1, D1), lambda i: (0, 0)),
            pl.BlockSpec((1, D1), lambda i: (0, 0)),
        ],
        out_specs=pl.BlockSpec((_BR, D1), lambda i: (i, 0)),
        out_shape=jax.ShapeDtypeStruct((NPAD, D1), jnp.float32),
    )(part2, xw2, dis2, b2r, Wh, bhr, mult, lo, hi)


def kernel(x_embeddings, edge_index, edge_weight, W1, b1, W2, b2,
           W_bi, b_bi, W_bd, b_bd, W_g, b_g):
    f32 = jnp.float32
    epad = EPT * NW - E_EDGES
    src2 = jnp.concatenate(
        [edge_index[0], jnp.zeros((epad,), jnp.int32)]).reshape(EROWS, CH)
    dst2 = jnp.concatenate(
        [edge_index[1], jnp.zeros((epad,), jnp.int32)]).reshape(EROWS, CH)
    ew2 = jnp.concatenate(
        [edge_weight, jnp.zeros((epad,), f32)]).reshape(EROWS, CH)
    x_pad = jnp.pad(x_embeddings, ((0, NPAD - N_NODES), (0, 0)))

    degp = _degree_partials(dst2, ew2)
    xw1, dis, dis2 = _tc1(x_pad, W1, degp.reshape(NC, NPAD, 1))
    dis_flat = dis.reshape(NPAD)

    part1 = _message_partials128(src2, dst2, ew2, dis_flat, xw1)
    xw2 = _tc2(part1, xw1, dis2, b1.reshape(1, D1), W2)
    src3 = src2.reshape(EROWS2, CH2)
    dst3 = dst2.reshape(EROWS2, CH2)
    ew3 = ew2.reshape(EROWS2, CH2)
    part2 = _message_partials(src3, dst3, ew3, dis_flat, xw2)

    Wh = jnp.zeros((D2, D1), f32)
    Wh = Wh.at[:, 0].set(W_bi[:, 0]).at[:, 1].set(W_bd[:, 0]).at[:, 2].set(W_g[:, 0])
    bh = jnp.zeros((1, D1), f32)
    bh = bh.at[0, 0].set(b_bi[0]).at[0, 1].set(b_bd[0]).at[0, 2].set(b_g[0])
    mult = jnp.zeros((1, D1), f32).at[0, 0].set(2.0).at[0, 1].set(0.1).at[0, 2].set(0.3)
    lo = jnp.zeros((1, D1), f32).at[0, 0].set(1e-4).at[0, 1].set(1e-6).at[0, 2].set(1e-4)
    hi = jnp.zeros((1, D1), f32).at[0, 0].set(2.0).at[0, 1].set(0.1).at[0, 2].set(0.3)

    outh = _tc3(part2, xw2, dis2, b2.reshape(1, D2), Wh, bh, mult, lo, hi)
    return (outh[:N_NODES, 0], outh[:N_NODES, 1], outh[:N_NODES, 2])


# speedup vs baseline: 1.0459x; 1.0459x over previous
import functools

import jax
import jax.numpy as jnp
from jax import lax
from jax.experimental import pallas as pl
from jax.experimental.pallas import tpu as pltpu
from jax.experimental.pallas import tpu_sc as plsc

N_NODES = 10000
NPAD = 10240
D1 = 128
D2 = 64
NC, NS, L = 2, 16, 16
NW = NC * NS
RPT = NPAD // NS
E_EDGES = 320000
CH = 128
EPT = ((E_EDGES // NW + CH * 8 - 1) // (CH * 8)) * (CH * 8)
NCHUNK = EPT // CH
EROWS = NW * NCHUNK
SB = 8

_MESH = dict(core_axis_name="c", subcore_axis_name="s", num_cores=NC,
             num_subcores=NS)
_SC_PARAMS = pltpu.CompilerParams(needs_layout_passes=False,
                                  use_tc_tiling_on_sc=False)


def _deg_body(dst_hbm, ew_hbm, degp_hbm, dst_v, ew_v, hist_v, red_v, slab_sh):
    c = lax.axis_index("c")
    s = lax.axis_index("s")
    wid = c * NS + s
    base = wid * NCHUNK
    pltpu.sync_copy(dst_hbm.at[pl.ds(base, NCHUNK)], dst_v)
    pltpu.sync_copy(ew_hbm.at[pl.ds(base, NCHUNK)], ew_v)

    def zero(i, _):
        hist_v[pl.ds(i * L, L)] = jnp.zeros((L,), jnp.float32)
        return 0
    lax.fori_loop(0, NPAD // L, zero, 0)

    def acc(i, _):
        for g in range(CH // L):
            dv = dst_v[i, pl.ds(g * L, L)]
            wv = ew_v[i, pl.ds(g * L, L)]
            plsc.addupdate_scatter(hist_v, [dv], wv)
        return 0
    lax.fori_loop(0, NCHUNK, acc, 0)

    pltpu.sync_copy(hist_v, slab_sh.at[s])
    plsc.subcore_barrier()
    pltpu.sync_copy(slab_sh.at[pl.ds(0, NS), pl.ds(s * RPT, RPT)], red_v)

    def red(i, _):
        v = red_v[0, pl.ds(i * L, L)]
        for p in range(1, NS):
            v = v + red_v[p, pl.ds(i * L, L)]
        hist_v[pl.ds(i * L, L)] = v
        return 0
    lax.fori_loop(0, RPT // L, red, 0)
    pltpu.sync_copy(hist_v.at[pl.ds(0, RPT)],
                    degp_hbm.at[c, pl.ds(s * RPT, RPT)])


def _degree_partials(dst2, ew2):
    return pl.kernel(
        _deg_body,
        out_type=jax.ShapeDtypeStruct((NC, NPAD), jnp.float32),
        mesh=plsc.VectorSubcoreMesh(**_MESH),
        compiler_params=_SC_PARAMS,
        scratch_types=[
            pltpu.VMEM((NCHUNK, CH), jnp.int32),
            pltpu.VMEM((NCHUNK, CH), jnp.float32),
            pltpu.VMEM((NPAD,), jnp.float32),
            pltpu.VMEM((NS, RPT), jnp.float32),
            pltpu.VMEM_SHARED((NS, NPAD), jnp.float32),
        ],
    )(dst2, ew2)


NSUP = NCHUNK // SB


CH2 = 256
NBC = EPT // CH2
EROWS2 = NW * NBC
CPS = 4
SB2 = CPS


def _msg_body(src_hbm, dst_hbm, ew_hbm, dis_hbm, xw_hbm,
              part_hbm, src_v, dst_v, ew_v, dis_v, nrm_v, rows0_v, rows1_v,
              g0, g1, s0, s1, acc_sh, xw_sh):
    D = D2
    c = lax.axis_index("c")
    s = lax.axis_index("s")
    wid = c * NS + s
    base = wid * NBC
    rows = (rows0_v, rows1_v)
    gsem = (g0, g1)
    ssem = (s0, s1)
    pltpu.sync_copy(dis_hbm, dis_v)
    pltpu.sync_copy(xw_hbm.at[pl.ds(s * RPT, RPT)],
                    xw_sh.at[pl.ds(s * RPT, RPT)])

    def zrow(i, _):
        for g in range(D // L):
            rows0_v[i, pl.ds(g * L, L)] = jnp.zeros((L,), jnp.float32)
        return 0
    lax.fori_loop(0, CH2, zrow, 0)
    for k in range(RPT // CH2):
        pltpu.sync_copy(rows0_v, acc_sh.at[pl.ds(s * RPT + k * CH2, CH2)])
    pltpu.sync_copy(rows0_v.at[pl.ds(0, RPT % CH2)],
                    acc_sh.at[pl.ds(s * RPT + (RPT // CH2) * CH2,
                                    RPT % CH2)])
    plsc.subcore_barrier()

    def stage(sjn):
        pltpu.sync_copy(src_hbm.at[pl.ds(base + sjn * SB2, SB2)], src_v)
        pltpu.sync_copy(ew_hbm.at[pl.ds(base + sjn * SB2, SB2)], ew_v)
        pltpu.sync_copy(dst_hbm.at[pl.ds(base + sjn * SB2, SB2)],
                        dst_v.at[sjn % 2])

    stage(0)
    pltpu.async_copy(xw_sh.at[src_v.at[0]], rows0_v, g0)

    def superchunk(sj, _):
        m = sj % 2
        for t in range(CPS):
            buf = t % 2
            nbuf = 1 - buf
            pltpu.make_async_copy(xw_sh.at[src_v.at[t]],
                                  rows[buf], gsem[buf]).wait()
            for g in range(CH2 // L):
                col = g * L
                sv = src_v[t, pl.ds(col, L)]
                dv = dst_v[m, t, pl.ds(col, L)]
                wv = ew_v[t, pl.ds(col, L)]
                nrm_v[pl.ds(g * L, L)] = (plsc.load_gather(dis_v, [sv]) * wv
                                          * plsc.load_gather(dis_v, [dv]))

            def _wait_prev_scatter():
                pltpu.make_async_copy(
                    rows[nbuf], acc_sh.at[dst_v.at[m, t]],
                    ssem[nbuf]).wait()
            if t == CPS - 1:
                @pl.when(sj + 1 < NSUP)
                def _():
                    stage(sj + 1)
                    _wait_prev_scatter()
                    pltpu.async_copy(xw_sh.at[src_v.at[0]],
                                     rows[nbuf], gsem[nbuf])
            elif t == 0:
                @pl.when(sj > 0)
                def _():
                    _wait_prev_scatter()
                pltpu.async_copy(xw_sh.at[src_v.at[t + 1]],
                                 rows[nbuf], gsem[nbuf])
            else:
                _wait_prev_scatter()
                pltpu.async_copy(xw_sh.at[src_v.at[t + 1]],
                                 rows[nbuf], gsem[nbuf])

            def scale(q, _):
                nv = nrm_v[pl.ds(q * L, L)]
                for tt in range(L):
                    sc = nv[tt]
                    e = q * L + tt
                    for g in range(D // L):
                        rows[buf][e, pl.ds(g * L, L)] = (
                            rows[buf][e, pl.ds(g * L, L)] * sc)
                return 0
            lax.fori_loop(0, CH2 // L, scale, 0)
            pltpu.async_copy(rows[buf],
                             acc_sh.at[dst_v.at[m, t]],
                             ssem[buf], add=True)
        return 0
    lax.fori_loop(0, NSUP, superchunk, 0)
    mlast = (NSUP - 1) % 2
    for b in range(2):
        pltpu.make_async_copy(
            rows[b], acc_sh.at[dst_v.at[mlast, CPS - 2 + b]],
            ssem[b]).wait()
    plsc.subcore_barrier()
    pltpu.sync_copy(acc_sh.at[pl.ds(s * RPT, RPT)],
                    part_hbm.at[c, pl.ds(s * RPT, RPT)])


def _message_partials(src2, dst2, ew2, dis, xw):
    return pl.kernel(
        _msg_body,
        out_type=jax.ShapeDtypeStruct((NC, NPAD, D2), jnp.float32),
        mesh=plsc.VectorSubcoreMesh(**_MESH),
        compiler_params=_SC_PARAMS,
        scratch_types=[
            pltpu.VMEM((SB2, CH2), jnp.int32),
            pltpu.VMEM((2, SB2, CH2), jnp.int32),
            pltpu.VMEM((SB2, CH2), jnp.float32),
            pltpu.VMEM((NPAD,), jnp.float32),
            pltpu.VMEM((CH2,), jnp.float32),
            pltpu.VMEM((CH2, D2), jnp.float32),
            pltpu.VMEM((CH2, D2), jnp.float32),
            pltpu.SemaphoreType.DMA,
            pltpu.SemaphoreType.DMA,
            pltpu.SemaphoreType.DMA,
            pltpu.SemaphoreType.DMA,
            pltpu.VMEM_SHARED((NPAD, D2), jnp.float32),
            pltpu.VMEM_SHARED((NPAD, D2), jnp.float32),
        ],
    )(src2, dst2, ew2, dis, xw)


_BR = 1024


def _tc1_body(x_ref, w_ref, degp_ref, xw_ref, xwa_ref, xwb_ref, dis_ref,
              dis2_ref):
    z = jnp.dot(x_ref[...], w_ref[...], preferred_element_type=jnp.float32)
    xw_ref[...] = z
    xwa_ref[...] = z[:, :D2]
    xwb_ref[...] = z[:, D2:]
    deg = degp_ref[0] + degp_ref[1] + 1.0
    d = jnp.where(deg > 0, lax.rsqrt(deg), 0.0)
    dis_ref[...] = d
    dis2_ref[...] = d * d


def _tc1(x_pad, W1, degp3):
    grid = (NPAD // _BR,)
    return pl.pallas_call(
        _tc1_body,
        grid=grid,
        in_specs=[
            pl.BlockSpec((_BR, D1), lambda i: (i, 0)),
            pl.BlockSpec((D1, D1), lambda i: (0, 0)),
            pl.BlockSpec((NC, _BR, 1), lambda i: (0, i, 0)),
        ],
        out_specs=[
            pl.BlockSpec((_BR, D1), lambda i: (i, 0)),
            pl.BlockSpec((_BR, D2), lambda i: (i, 0)),
            pl.BlockSpec((_BR, D2), lambda i: (i, 0)),
            pl.BlockSpec((_BR, 1), lambda i: (i, 0)),
            pl.BlockSpec((_BR, 1), lambda i: (i, 0)),
        ],
        out_shape=[
            jax.ShapeDtypeStruct((NPAD, D1), jnp.float32),
            jax.ShapeDtypeStruct((NPAD, D2), jnp.float32),
            jax.ShapeDtypeStruct((NPAD, D2), jnp.float32),
            jax.ShapeDtypeStruct((NPAD, 1), jnp.float32),
            jax.ShapeDtypeStruct((NPAD, 1), jnp.float32),
        ],
    )(x_pad, W1, degp3)


def _tc2_body(pa_ref, pb_ref, xw_ref, dis2_ref, b_ref, w2_ref, xw2_ref):
    p = jnp.concatenate([pa_ref[0] + pa_ref[1], pb_ref[0] + pb_ref[1]],
                        axis=-1)
    x1 = p + xw_ref[...] * dis2_ref[...] + b_ref[...]
    x1 = jnp.maximum(x1, 0.0)
    xw2_ref[...] = jnp.dot(x1, w2_ref[...], preferred_element_type=jnp.float32)


def _tc2(parta, partb, xw1, dis2, b1r, W2):
    grid = (NPAD // _BR,)
    return pl.pallas_call(
        _tc2_body,
        grid=grid,
        in_specs=[
            pl.BlockSpec((NC, _BR, D2), lambda i: (0, i, 0)),
            pl.BlockSpec((NC, _BR, D2), lambda i: (0, i, 0)),
            pl.BlockSpec((_BR, D1), lambda i: (i, 0)),
            pl.BlockSpec((_BR, 1), lambda i: (i, 0)),
            pl.BlockSpec((1, D1), lambda i: (0, 0)),
            pl.BlockSpec((D1, D2), lambda i: (0, 0)),
        ],
        out_specs=pl.BlockSpec((_BR, D2), lambda i: (i, 0)),
        out_shape=jax.ShapeDtypeStruct((NPAD, D2), jnp.float32),
    )(parta, partb, xw1, dis2, b1r, W2)


def _tc3_body(p_ref, xw_ref, dis2_ref, b_ref, wh_ref, bh_ref, mult_ref,
              lo_ref, hi_ref, out_ref):
    x2 = p_ref[0] + p_ref[1] + xw_ref[...] * dis2_ref[...] + b_ref[...]
    x2 = jnp.maximum(x2, 0.0)
    z = jnp.dot(x2, wh_ref[...], preferred_element_type=jnp.float32)
    z = z + bh_ref[...]
    h = 1.0 / (1.0 + jnp.exp(-z))
    out_ref[...] = jnp.clip(h * mult_ref[...], lo_ref[...], hi_ref[...])


def _tc3(part2, xw2, dis2, b2r, Wh, bhr, mult, lo, hi):
    grid = (NPAD // _BR,)
    return pl.pallas_call(
        _tc3_body,
        grid=grid,
        in_specs=[
            pl.BlockSpec((NC, _BR, D2), lambda i: (0, i, 0)),
            pl.BlockSpec((_BR, D2), lambda i: (i, 0)),
            pl.BlockSpec((_BR, 1), lambda i: (i, 0)),
            pl.BlockSpec((1, D2), lambda i: (0, 0)),
            pl.BlockSpec((D2, D1), lambda i: (0, 0)),
            pl.BlockSpec((1, D1), lambda i: (0, 0)),
            pl.BlockSpec((1, D1), lambda i: (0, 0)),
            pl.BlockSpec((1, D1), lambda i: (0, 0)),
            pl.BlockSpec((1, D1), lambda i: (0, 0)),
        ],
        out_specs=pl.BlockSpec((_BR, D1), lambda i: (i, 0)),
        out_shape=jax.ShapeDtypeStruct((NPAD, D1), jnp.float32),
    )(part2, xw2, dis2, b2r, Wh, bhr, mult, lo, hi)


def kernel(x_embeddings, edge_index, edge_weight, W1, b1, W2, b2,
           W_bi, b_bi, W_bd, b_bd, W_g, b_g):
    f32 = jnp.float32
    epad = EPT * NW - E_EDGES
    src2 = jnp.concatenate(
        [edge_index[0], jnp.zeros((epad,), jnp.int32)]).reshape(EROWS, CH)
    dst2 = jnp.concatenate(
        [edge_index[1], jnp.zeros((epad,), jnp.int32)]).reshape(EROWS, CH)
    ew2 = jnp.concatenate(
        [edge_weight, jnp.zeros((epad,), f32)]).reshape(EROWS, CH)
    x_pad = jnp.pad(x_embeddings, ((0, NPAD - N_NODES), (0, 0)))

    degp = _degree_partials(dst2, ew2)
    xw1, xw1a, xw1b, dis, dis2 = _tc1(x_pad, W1, degp.reshape(NC, NPAD, 1))
    dis_flat = dis.reshape(NPAD)

    src3 = src2.reshape(EROWS2, CH2)
    dst3 = dst2.reshape(EROWS2, CH2)
    ew3 = ew2.reshape(EROWS2, CH2)
    parta = _message_partials(src3, dst3, ew3, dis_flat, xw1a)
    partb = _message_partials(src3, dst3, ew3, dis_flat, xw1b)
    xw2 = _tc2(parta, partb, xw1, dis2, b1.reshape(1, D1), W2)
    part2 = _message_partials(src3, dst3, ew3, dis_flat, xw2)

    Wh = jnp.zeros((D2, D1), f32)
    Wh = Wh.at[:, 0].set(W_bi[:, 0]).at[:, 1].set(W_bd[:, 0]).at[:, 2].set(W_g[:, 0])
    bh = jnp.zeros((1, D1), f32)
    bh = bh.at[0, 0].set(b_bi[0]).at[0, 1].set(b_bd[0]).at[0, 2].set(b_g[0])
    mult = jnp.zeros((1, D1), f32).at[0, 0].set(2.0).at[0, 1].set(0.1).at[0, 2].set(0.3)
    lo = jnp.zeros((1, D1), f32).at[0, 0].set(1e-4).at[0, 1].set(1e-6).at[0, 2].set(1e-4)
    hi = jnp.zeros((1, D1), f32).at[0, 0].set(2.0).at[0, 1].set(0.1).at[0, 2].set(0.3)

    outh = _tc3(part2, xw2, dis2, b2.reshape(1, D2), Wh, bh, mult, lo, hi)
    return (outh[:N_NODES, 0], outh[:N_NODES, 1], outh[:N_NODES, 2])
